# baseline (speedup 1.0000x reference)
import functools

import jax
import jax.numpy as jnp
from jax import lax
from jax.experimental import pallas as pl
from jax.experimental.pallas import tpu as pltpu
from jax.experimental.pallas import tpu_sc as plsc

B = 4096
L = 200
VOCAB = 1000000
D = 64

NW = 32
TOTAL = B * L
ROWS128 = TOTAL // 128
ROWS_PER_W = ROWS128 // NW
CHUNK_ROWS = 8
NCHUNKS = ROWS_PER_W // CHUNK_ROWS

_mesh = plsc.VectorSubcoreMesh(core_axis_name="c", subcore_axis_name="s")


@functools.partial(
    pl.kernel,
    mesh=_mesh,
    out_type=jax.ShapeDtypeStruct((ROWS128, 128, D), jnp.float32),
    scratch_types=[
        pltpu.VMEM((CHUNK_ROWS, 128), jnp.int32),
        pltpu.VMEM((CHUNK_ROWS, 128, D), jnp.float32),
        pltpu.SemaphoreType.DMA,
    ],
    compiler_params=pltpu.CompilerParams(use_tc_tiling_on_sc=False),
)
def _gather_kernel(tok_hbm, table_hbm, out_hbm, idx_v, rows_v, sem):
    wid = lax.axis_index("s") * 2 + lax.axis_index("c")
    base_row = wid * ROWS_PER_W

    def body(g, _):
        row0 = base_row + g * CHUNK_ROWS
        pltpu.sync_copy(tok_hbm.at[pl.ds(row0, CHUNK_ROWS)], idx_v)
        copies = [
            pltpu.async_copy(table_hbm.at[idx_v.at[j]], rows_v.at[j], sem)
            for j in range(CHUNK_ROWS)
        ]
        for c in copies:
            c.wait()
        pltpu.sync_copy(rows_v, out_hbm.at[pl.ds(row0, CHUNK_ROWS)])
        return 0

    lax.fori_loop(0, NCHUNKS, body, 0)


def kernel(tokens, word_embed_weight):
    tok = tokens.reshape(ROWS128, 128).astype(jnp.int32)
    out = _gather_kernel(tok, word_embed_weight)
    return out.reshape(B, L, D)


# speedup vs baseline: 1.0082x; 1.0082x over previous
import functools

import jax
import jax.numpy as jnp
from jax import lax
from jax.experimental import pallas as pl
from jax.experimental.pallas import tpu as pltpu
from jax.experimental.pallas import tpu_sc as plsc

B = 4096
L = 200
VOCAB = 1000000
D = 64

NW = 32
TOTAL = B * L
ROWS128 = TOTAL // 128
ROWS_PER_W = ROWS128 // NW
CHUNK_ROWS = 5
NCHUNKS = ROWS_PER_W // CHUNK_ROWS
NBUF = 2

_mesh = plsc.VectorSubcoreMesh(core_axis_name="c", subcore_axis_name="s")


@functools.partial(
    pl.kernel,
    mesh=_mesh,
    out_type=jax.ShapeDtypeStruct((ROWS128, 128, D), jnp.float32),
    scratch_types=[
        pltpu.VMEM((NBUF, CHUNK_ROWS, 128), jnp.int32),
        pltpu.VMEM((NBUF, CHUNK_ROWS, 128, D), jnp.float32),
        pltpu.SemaphoreType.DMA,
        pltpu.SemaphoreType.DMA,
        pltpu.SemaphoreType.DMA,
        pltpu.SemaphoreType.DMA,
    ],
    compiler_params=pltpu.CompilerParams(use_tc_tiling_on_sc=False),
)
def _gather_kernel(tok_hbm, table_hbm, out_hbm, idx_v, rows_v,
                   gsem0, gsem1, ssem0, ssem1):
    wid = lax.axis_index("s") * 2 + lax.axis_index("c")
    base_row = wid * ROWS_PER_W
    gsem = (gsem0, gsem1)
    ssem = (ssem0, ssem1)

    def fire(g, b):
        row0 = base_row + g * CHUNK_ROWS
        pltpu.sync_copy(tok_hbm.at[pl.ds(row0, CHUNK_ROWS)], idx_v.at[b])
        for j in range(CHUNK_ROWS):
            pltpu.async_copy(table_hbm.at[idx_v.at[b, j]], rows_v.at[b, j],
                             gsem[b])

    def wait_gathers(b):
        pltpu.make_async_copy(out_hbm.at[pl.ds(0, CHUNK_ROWS)],
                              rows_v.at[b], gsem[b]).wait()

    def fire_store(g, b):
        row0 = base_row + g * CHUNK_ROWS
        pltpu.async_copy(rows_v.at[b], out_hbm.at[pl.ds(row0, CHUNK_ROWS)],
                         ssem[b])

    def wait_store(b):
        pltpu.make_async_copy(rows_v.at[b], out_hbm.at[pl.ds(0, CHUNK_ROWS)],
                              ssem[b]).wait()

    fire(0, 0)
    fire(1, 1)
    wait_gathers(0)
    fire_store(0, 0)

    def body(k, _):
        g0 = 2 + 2 * k
        for b in range(NBUF):
            g = g0 + b
            wait_store(b)
            fire(g, b)
            wait_gathers(b ^ 1)
            fire_store(g - 1, b ^ 1)
        return 0

    lax.fori_loop(0, (NCHUNKS - 2) // 2, body, 0)

    b_last = (NCHUNKS - 1) % 2
    wait_gathers(b_last)
    fire_store(NCHUNKS - 1, b_last)
    wait_store(0)
    wait_store(1)


def kernel(tokens, word_embed_weight):
    tok = tokens.reshape(ROWS128, 128).astype(jnp.int32)
    out = _gather_kernel(tok, word_embed_weight)
    return out.reshape(B, L, D)


# speedup vs baseline: 1.3428x; 1.3319x over previous
import functools

import jax
import jax.numpy as jnp
from jax import lax
from jax.experimental import pallas as pl
from jax.experimental.pallas import tpu as pltpu
from jax.experimental.pallas import tpu_sc as plsc

B = 4096
L = 200
VOCAB = 1000000
D = 64

NW = 32
TOTAL = B * L
ROWS128 = TOTAL // 128
ROWS_PER_W = ROWS128 // NW
CHUNK_ROWS = 5
NCHUNKS = ROWS_PER_W // CHUNK_ROWS
NBUF = 2

_mesh = plsc.VectorSubcoreMesh(core_axis_name="c", subcore_axis_name="s")


@functools.partial(
    pl.kernel,
    mesh=_mesh,
    out_type=jax.ShapeDtypeStruct((ROWS128, 128, 128), jnp.float32),
    scratch_types=[
        pltpu.VMEM((NBUF, CHUNK_ROWS, 128), jnp.int32),
        pltpu.VMEM((NBUF, CHUNK_ROWS, 128, D), jnp.float32),
        pltpu.SemaphoreType.DMA,
        pltpu.SemaphoreType.DMA,
        pltpu.SemaphoreType.DMA,
        pltpu.SemaphoreType.DMA,
    ],
    compiler_params=pltpu.CompilerParams(use_tc_tiling_on_sc=False),
)
def _gather_kernel(tok_hbm, table_hbm, out_hbm, idx_v, rows_v,
                   gsem0, gsem1, ssem0, ssem1):
    wid = lax.axis_index("s") * 2 + lax.axis_index("c")
    base_row = wid * ROWS_PER_W
    gsem = (gsem0, gsem1)
    ssem = (ssem0, ssem1)

    def fire(g, b):
        row0 = base_row + g * CHUNK_ROWS
        pltpu.sync_copy(tok_hbm.at[pl.ds(row0, CHUNK_ROWS)], idx_v.at[b])
        for j in range(CHUNK_ROWS):
            pltpu.async_copy(table_hbm.at[idx_v.at[b, j]], rows_v.at[b, j],
                             gsem[b])

    def wait_gathers(b):
        pltpu.make_async_copy(out_hbm.at[pl.ds(0, CHUNK_ROWS)],
                              rows_v.at[b], gsem[b]).wait()

    def fire_store(g, b):
        row0 = base_row + g * CHUNK_ROWS
        pltpu.async_copy(rows_v.at[b],
                         out_hbm.at[pl.ds(row0, CHUNK_ROWS), :, pl.ds(0, D)],
                         ssem[b])

    def wait_store(b):
        pltpu.make_async_copy(
            rows_v.at[b],
            out_hbm.at[pl.ds(0, CHUNK_ROWS), :, pl.ds(0, D)],
            ssem[b]).wait()

    fire(0, 0)
    fire(1, 1)
    wait_gathers(0)
    fire_store(0, 0)

    def body(k, _):
        g0 = 2 + 2 * k
        for b in range(NBUF):
            g = g0 + b
            wait_store(b)
            fire(g, b)
            wait_gathers(b ^ 1)
            fire_store(g - 1, b ^ 1)
        return 0

    lax.fori_loop(0, (NCHUNKS - 2) // 2, body, 0)

    b_last = (NCHUNKS - 1) % 2
    wait_gathers(b_last)
    fire_store(NCHUNKS - 1, b_last)
    wait_store(0)
    wait_store(1)


def kernel(tokens, word_embed_weight):
    tok = tokens.reshape(ROWS128, 128).astype(jnp.int32)
    out = _gather_kernel(tok, word_embed_weight)
    return out.reshape(TOTAL, 128)[:, :D].reshape(B, L, D)
